# baseline (speedup 1.0000x reference)
import jax
import jax.numpy as jnp
from jax import lax
from jax.experimental import pallas as pl
from jax.experimental.pallas import tpu as pltpu
from jax.experimental.pallas import tpu_sc as plsc

N_TABLES = 26
NUM_ROWS = 100000
DIM = 64
BATCH = 1024
BAG = 20

NW = 32
BAGS_PW = BATCH // NW
IDX_PW = BAGS_PW * BAG
IDX_ALL = N_TABLES * IDX_PW
ROWS_PS = IDX_PW // 2
BAGS_PS = BAGS_PW // 2
NSTEP = 2 * N_TABLES
CHUNKS = (128, 128, 64)


def _body(idx_hbm, wt_hbm, out_hbm, idx_v, pcol_v, rows_v, out_v, sem):
    cid = lax.axis_index("c")
    sid = lax.axis_index("s")
    wid = sid * 2 + cid

    ibase = pl.multiple_of(wid * IDX_ALL, 8)
    pltpu.sync_copy(idx_hbm.at[pl.ds(ibase, IDX_ALL)], idx_v)

    def lin_step(t, carry):
        off = (t * NUM_ROWS).astype(jnp.int32)
        tb = t * IDX_PW
        for c in range(IDX_PW // 16):
            s = pl.ds(tb + c * 16, 16)
            raw = idx_v[s]
            pcol_v[s] = (raw & 1).astype(jnp.float32)
            idx_v[s] = lax.shift_right_logical(raw + off, 1)
        return carry

    lax.fori_loop(0, N_TABLES, lin_step, 0)

    def g_copies(s, par):
        out = []
        o = 0
        for n in CHUNKS:
            out.append((wt_hbm.at[idx_v.at[pl.ds(s * ROWS_PS + o, n)]],
                        rows_v.at[pl.ds(par * ROWS_PS + o, n)]))
            o += n
        return out

    def fire(s, par):
        for src, dst in g_copies(s, par):
            pltpu.async_copy(src, dst, sem)

    def drain(s, par):
        for src, dst in g_copies(s, par):
            pltpu.make_async_copy(src, dst, sem).wait()

    fire(0, 0)

    def s_step(s, carry):
        par = s % 2
        drain(s, par)

        @pl.when(s + 1 < NSTEP)
        def _():
            fire(s + 1, 1 - par)

        rbase = par * ROWS_PS
        sb0 = s * ROWS_PS
        GB = 4
        GR = GB * BAG

        def group_step(g, carry2):
            pv = [pcol_v[pl.ds(sb0 + g * GR + j * 16, 16)]
                  for j in range(GR // 16)]
            pfs = [
                jnp.broadcast_to(pv[kk // 16][kk % 16], (16,))
                for kk in range(GR)
            ]
            for bb in range(GB):
                rb = rbase + g * GR + bb * BAG
                for c in range(DIM // 16):
                    lo = pl.ds(c * 16, 16)
                    hi = pl.ds(DIM + c * 16, 16)
                    acc = jnp.zeros((16,), jnp.float32)
                    for k in range(BAG):
                        a = rows_v[rb + k, lo]
                        d = rows_v[rb + k, hi] - a
                        acc = acc + (a + d * pfs[bb * BAG + k])
                    out_v[g * GB + bb, lo] = acc
            return carry2

        lax.fori_loop(0, BAGS_PS // GB, group_step, 0)
        t = s // 2
        obase = wid * BAGS_PW + (s % 2) * BAGS_PS
        pltpu.sync_copy(out_v, out_hbm.at[t].at[pl.ds(obase, BAGS_PS)])
        return carry

    lax.fori_loop(0, NSTEP, s_step, 0)


@jax.jit
def _run(idx_wm, wt_pairs):
    mesh = plsc.VectorSubcoreMesh(core_axis_name="c", subcore_axis_name="s")
    f = pl.kernel(
        _body,
        out_type=jax.ShapeDtypeStruct((N_TABLES, BATCH, DIM), jnp.float32),
        mesh=mesh,
        scratch_types=[
            pltpu.VMEM((IDX_ALL,), jnp.int32),
            pltpu.VMEM((IDX_ALL,), jnp.float32),
            pltpu.VMEM((2 * ROWS_PS, 2 * DIM), jnp.float32),
            pltpu.VMEM((BAGS_PS, DIM), jnp.float32),
            pltpu.SemaphoreType.DMA,
        ],
    )
    return f(idx_wm, wt_pairs)


def kernel(indices, weights):
    idx_wm = (indices.astype(jnp.int32)
              .reshape(N_TABLES, NW, BAGS_PW * BAG)
              .transpose(1, 0, 2)
              .reshape(N_TABLES * BATCH * BAG))
    wt_pairs = weights.reshape(N_TABLES * NUM_ROWS // 2, 2 * DIM)
    return _run(idx_wm, wt_pairs)


# speedup vs baseline: 1.0208x; 1.0208x over previous
import jax
import jax.numpy as jnp
from jax import lax
from jax.experimental import pallas as pl
from jax.experimental.pallas import tpu as pltpu
from jax.experimental.pallas import tpu_sc as plsc

N_TABLES = 26
NUM_ROWS = 100000
DIM = 64
BATCH = 1024
BAG = 20

NW = 32
BAGS_PW = BATCH // NW
IDX_PW = BAGS_PW * BAG
IDX_ALL = N_TABLES * IDX_PW


def _body(idx_hbm, wt_hbm, out_hbm, idx_v, rows_v, out_v, sem):
    cid = lax.axis_index("c")
    sid = lax.axis_index("s")
    wid = sid * 2 + cid

    ibase = pl.multiple_of(wid * IDX_ALL, 8)
    pltpu.sync_copy(idx_hbm.at[pl.ds(ibase, IDX_ALL)], idx_v)

    def lin_step(t, carry):
        off = (t * NUM_ROWS).astype(jnp.int32)
        tb = t * IDX_PW
        for c in range(IDX_PW // 16):
            s = pl.ds(tb + c * 16, 16)
            idx_v[s] = idx_v[s] + off
        return carry

    lax.fori_loop(0, N_TABLES, lin_step, 0)

    def g_copy(t, par):
        return (wt_hbm.at[idx_v.at[pl.ds(t * IDX_PW, IDX_PW)]],
                rows_v.at[pl.ds(par * IDX_PW, IDX_PW)])

    fire0 = g_copy(0, 0)
    pltpu.async_copy(fire0[0], fire0[1], sem)

    def t_step(t, carry):
        par = t % 2
        src, dst = g_copy(t, par)
        pltpu.make_async_copy(src, dst, sem).wait()

        @pl.when(t + 1 < N_TABLES)
        def _():
            src2, dst2 = g_copy(t + 1, 1 - par)
            pltpu.async_copy(src2, dst2, sem)

        def bag_step(b, carry2):
            rb = par * IDX_PW + b * BAG
            for c in range(DIM // 16):
                s = pl.ds(c * 16, 16)
                acc = rows_v[rb, s]
                for k in range(1, BAG):
                    acc = acc + rows_v[rb + k, s]
                out_v[b, s] = acc
            return carry2

        lax.fori_loop(0, BAGS_PW, bag_step, 0)
        pltpu.sync_copy(out_v,
                        out_hbm.at[t].at[pl.ds(wid * BAGS_PW, BAGS_PW)])
        return carry

    lax.fori_loop(0, N_TABLES, t_step, 0)


@jax.jit
def _run(idx_wm, wt_merged):
    mesh = plsc.VectorSubcoreMesh(core_axis_name="c", subcore_axis_name="s")
    f = pl.kernel(
        _body,
        out_type=jax.ShapeDtypeStruct((N_TABLES, BATCH, DIM), jnp.float32),
        mesh=mesh,
        scratch_types=[
            pltpu.VMEM((IDX_ALL,), jnp.int32),
            pltpu.VMEM((2 * IDX_PW, DIM), jnp.float32),
            pltpu.VMEM((BAGS_PW, DIM), jnp.float32),
            pltpu.SemaphoreType.DMA,
        ],
        compiler_params=pltpu.CompilerParams(use_tc_tiling_on_sc=False),
    )
    return f(idx_wm, wt_merged)


def kernel(indices, weights):
    idx_wm = (indices.astype(jnp.int32)
              .reshape(N_TABLES, NW, BAGS_PW * BAG)
              .transpose(1, 0, 2)
              .reshape(N_TABLES * BATCH * BAG))
    wt_merged = weights.reshape(N_TABLES * NUM_ROWS, DIM)
    return _run(idx_wm, wt_merged)
